# baseline (speedup 1.0000x reference)
import jax
import jax.numpy as jnp
from jax.experimental import pallas as pl
from jax.experimental.pallas import tpu as pltpu

_ALPHA = 1.702
_LIMIT = 7.0
_E = 8
_TT = 512


def _moe_dense_kernel(wr_ref, br_ref, x_ref, wg_ref, bg_ref, wd_ref, bd_ref,
                      out_ref, e1_s, e2_s, w1_s, w2_s, wgb_s, wdb_s):
    e = pl.program_id(0)
    t = pl.program_id(1)
    d_ff = wd_ref.shape[2]
    x = x_ref[pl.ds(t * _TT, _TT), :]

    @pl.when(t == 0)
    def _cast_weights():
        wgb_s[...] = wg_ref[0].astype(jnp.bfloat16)
        wdb_s[...] = wd_ref[0].astype(jnp.bfloat16)

    @pl.when(e == 0)
    def _router():
        logits = jax.lax.dot_general(
            x, wr_ref[...], (((1,), (1,)), ((), ())),
            preferred_element_type=jnp.float32)
        logits = logits + br_ref[...]
        iota = jax.lax.broadcasted_iota(jnp.int32, logits.shape, 1)
        m1 = jnp.max(logits, axis=1, keepdims=True)
        e1 = jnp.min(jnp.where(logits == m1, iota, _E), axis=1, keepdims=True)
        l2 = jnp.where(iota == e1, -jnp.inf, logits)
        m2 = jnp.max(l2, axis=1, keepdims=True)
        e2 = jnp.min(jnp.where(l2 == m2, iota, _E), axis=1, keepdims=True)
        w1 = 1.0 / (1.0 + jnp.exp(m2 - m1))
        w2 = 1.0 / (1.0 + jnp.exp(m1 - m2))
        e1_s[pl.ds(t * _TT, _TT), :] = e1
        e2_s[pl.ds(t * _TT, _TT), :] = e2
        w1_s[pl.ds(t * _TT, _TT), :] = w1
        w2_s[pl.ds(t * _TT, _TT), :] = w2

    gu = jax.lax.dot_general(
        x.astype(jnp.bfloat16), wgb_s[...], (((1,), (1,)), ((), ())),
        preferred_element_type=jnp.float32)
    gu = gu + bg_ref[0]
    gate = jnp.minimum(gu[:, :d_ff], _LIMIT)
    up = jnp.clip(gu[:, d_ff:], -_LIMIT, _LIMIT)
    act = (up + 1.0) * (gate * jax.nn.sigmoid(_ALPHA * gate))
    eo = jax.lax.dot_general(
        act.astype(jnp.bfloat16), wdb_s[...], (((1,), (1,)), ((), ())),
        preferred_element_type=jnp.float32)
    eo = eo + bd_ref[0]

    e1 = e1_s[pl.ds(t * _TT, _TT), :]
    e2 = e2_s[pl.ds(t * _TT, _TT), :]
    w1 = w1_s[pl.ds(t * _TT, _TT), :]
    w2 = w2_s[pl.ds(t * _TT, _TT), :]
    w = jnp.where(e1 == e, w1, 0.0) + jnp.where(e2 == e, w2, 0.0)
    val = w * eo

    @pl.when(e == 0)
    def _init():
        out_ref[pl.ds(t * _TT, _TT), :] = val

    @pl.when(e != 0)
    def _acc():
        out_ref[pl.ds(t * _TT, _TT), :] += val


def kernel(x, router_weight, router_bias, gate_up_proj, gate_up_proj_bias,
           down_proj, down_proj_bias):
    T, D = x.shape
    E, I2, _ = gate_up_proj.shape
    d_ff = I2 // 2
    nt = T // _TT
    br2d = router_bias.reshape(1, E)
    bg3d = gate_up_proj_bias.reshape(E, 1, I2)
    bd3d = down_proj_bias.reshape(E, 1, D)

    out = pl.pallas_call(
        _moe_dense_kernel,
        grid=(E, nt),
        in_specs=[
            pl.BlockSpec((E, D), lambda e, t: (0, 0)),
            pl.BlockSpec((1, E), lambda e, t: (0, 0)),
            pl.BlockSpec((T, D), lambda e, t: (0, 0)),
            pl.BlockSpec((1, I2, D), lambda e, t: (e, 0, 0)),
            pl.BlockSpec((1, 1, I2), lambda e, t: (e, 0, 0)),
            pl.BlockSpec((1, D, d_ff), lambda e, t: (e, 0, 0)),
            pl.BlockSpec((1, 1, D), lambda e, t: (e, 0, 0)),
        ],
        out_specs=pl.BlockSpec((T, D), lambda e, t: (0, 0)),
        out_shape=jax.ShapeDtypeStruct((T, D), jnp.float32),
        scratch_shapes=[
            pltpu.VMEM((T, 1), jnp.int32),
            pltpu.VMEM((T, 1), jnp.int32),
            pltpu.VMEM((T, 1), jnp.float32),
            pltpu.VMEM((T, 1), jnp.float32),
            pltpu.VMEM((I2, D), jnp.bfloat16),
            pltpu.VMEM((D, d_ff), jnp.bfloat16),
        ],
        compiler_params=pltpu.CompilerParams(
            dimension_semantics=("arbitrary", "arbitrary"),
        ),
    )(router_weight, br2d, x, gate_up_proj, bg3d, down_proj, bd3d)
    return out


# speedup vs baseline: 1.1428x; 1.1428x over previous
import jax
import jax.numpy as jnp
from jax.experimental import pallas as pl
from jax.experimental.pallas import tpu as pltpu

_ALPHA = 1.702
_LIMIT = 7.0
_E = 8
_TT = 1024


def _moe_dense_kernel(wr_ref, br_ref, x_ref, wg_ref, bg_ref, wd_ref, bd_ref,
                      out_ref, e1_s, e2_s, w1_s, w2_s):
    e = pl.program_id(0)
    t = pl.program_id(1)
    d_ff = wd_ref.shape[2]
    x = x_ref[pl.ds(t * _TT, _TT), :]

    @pl.when(e == 0)
    def _router():
        logits = jax.lax.dot_general(
            x, wr_ref[...], (((1,), (1,)), ((), ())),
            preferred_element_type=jnp.float32)
        logits = logits + br_ref[...]
        iota = jax.lax.broadcasted_iota(jnp.int32, logits.shape, 1)
        m1 = jnp.max(logits, axis=1, keepdims=True)
        e1 = jnp.min(jnp.where(logits == m1, iota, _E), axis=1, keepdims=True)
        l2 = jnp.where(iota == e1, -jnp.inf, logits)
        m2 = jnp.max(l2, axis=1, keepdims=True)
        e2 = jnp.min(jnp.where(l2 == m2, iota, _E), axis=1, keepdims=True)
        w1 = 1.0 / (1.0 + jnp.exp(m2 - m1))
        w2 = 1.0 / (1.0 + jnp.exp(m1 - m2))
        e1_s[pl.ds(t * _TT, _TT), :] = e1
        e2_s[pl.ds(t * _TT, _TT), :] = e2
        w1_s[pl.ds(t * _TT, _TT), :] = w1
        w2_s[pl.ds(t * _TT, _TT), :] = w2

    gu = jax.lax.dot_general(
        x, wg_ref[0], (((1,), (1,)), ((), ())),
        preferred_element_type=jnp.float32)
    gu = gu + bg_ref[0]
    gate = jnp.minimum(gu[:, :d_ff], _LIMIT)
    up = jnp.clip(gu[:, d_ff:], -_LIMIT, _LIMIT)
    act = (up + 1.0) * (gate * jax.nn.sigmoid(_ALPHA * gate))
    eo = jax.lax.dot_general(
        act, wd_ref[0], (((1,), (1,)), ((), ())),
        preferred_element_type=jnp.float32)
    eo = eo + bd_ref[0]

    e1 = e1_s[pl.ds(t * _TT, _TT), :]
    e2 = e2_s[pl.ds(t * _TT, _TT), :]
    w1 = w1_s[pl.ds(t * _TT, _TT), :]
    w2 = w2_s[pl.ds(t * _TT, _TT), :]
    w = jnp.where(e1 == e, w1, 0.0) + jnp.where(e2 == e, w2, 0.0)
    val = w * eo

    @pl.when(e == 0)
    def _init():
        out_ref[pl.ds(t * _TT, _TT), :] = val

    @pl.when(e != 0)
    def _acc():
        out_ref[pl.ds(t * _TT, _TT), :] += val


def kernel(x, router_weight, router_bias, gate_up_proj, gate_up_proj_bias,
           down_proj, down_proj_bias):
    T, D = x.shape
    E, I2, _ = gate_up_proj.shape
    d_ff = I2 // 2
    nt = T // _TT
    br2d = router_bias.reshape(1, E)
    bg3d = gate_up_proj_bias.reshape(E, 1, I2)
    bd3d = down_proj_bias.reshape(E, 1, D)

    out = pl.pallas_call(
        _moe_dense_kernel,
        grid=(E, nt),
        in_specs=[
            pl.BlockSpec((E, D), lambda e, t: (0, 0)),
            pl.BlockSpec((1, E), lambda e, t: (0, 0)),
            pl.BlockSpec((T, D), lambda e, t: (0, 0)),
            pl.BlockSpec((1, I2, D), lambda e, t: (e, 0, 0)),
            pl.BlockSpec((1, 1, I2), lambda e, t: (e, 0, 0)),
            pl.BlockSpec((1, D, d_ff), lambda e, t: (e, 0, 0)),
            pl.BlockSpec((1, 1, D), lambda e, t: (e, 0, 0)),
        ],
        out_specs=pl.BlockSpec((T, D), lambda e, t: (0, 0)),
        out_shape=jax.ShapeDtypeStruct((T, D), jnp.float32),
        scratch_shapes=[
            pltpu.VMEM((T, 1), jnp.int32),
            pltpu.VMEM((T, 1), jnp.int32),
            pltpu.VMEM((T, 1), jnp.float32),
            pltpu.VMEM((T, 1), jnp.float32),
        ],
        compiler_params=pltpu.CompilerParams(
            dimension_semantics=("arbitrary", "arbitrary"),
        ),
    )(router_weight, br2d, x, gate_up_proj, bg3d, down_proj, bd3d)
    return out
